# baseline (speedup 1.0000x reference)
import numpy as np
import jax
import jax.numpy as jnp
from jax.experimental import pallas as pl

H = 256
W = 256
STRIP = 32
NSTRIP = H // STRIP
PAD = 8
PH = H + 2 * PAD
BAND = STRIP + 2 * PAD
PR = STRIP + 2
QR = PR + 4

def _spiral_coords(sr):
    coords = [(0, 0)]
    y = x = 0
    moves = [(0, 1), (1, 0), (0, -1), (-1, 0)]
    step = 1
    d = 0
    sz = 2 * sr + 1
    while len(coords) < sz * sz:
        for _ in range(2):
            dy, dx = moves[d % 4]
            for _ in range(step):
                y += dy
                x += dx
                if abs(y) <= sr and abs(x) <= sr and len(coords) < sz * sz:
                    coords.append((y, x))
            d += 1
        step += 1
    return coords


_SPIRAL = _spiral_coords(3)

_C5_NP = np.zeros((260, 256), np.float32)
_C3_NP = np.zeros((260, 256), np.float32)
for _i in range(260):
    for _j in range(256):
        if 0 <= _i - _j <= 4:
            _C5_NP[_i, _j] = 1.0
        if 1 <= _i - _j <= 3:
            _C3_NP[_i, _j] = 1.0
_C53_NP = np.concatenate([_C5_NP, _C3_NP], axis=1)

_DN = (((1,), (0,)), ((), ()))


def _flow_kernel(fp_ref, gp_ref, c5b_ref, c53b_ref, out_ref):
    pid = pl.program_id(0)
    r0 = pid * STRIP
    fpad = fp_ref[pl.ds(r0, BAND), :]
    gpad = gp_ref[pl.ds(r0, BAND), :]

    def smooth_q(x):
        v = (x[0:BAND - 2, :] + 2.0 * x[1:BAND - 1, :] + x[2:BAND, :]) * 0.25
        h = (v[:, 0:270] + 2.0 * v[:, 1:271] + v[:, 2:272]) * 0.25
        return jnp.clip(jnp.round(h * 255.0), 0.0, 255.0)

    zrow = jnp.zeros((1, 270), jnp.float32)
    zcol = jnp.zeros((BAND, 1), jnp.float32)

    def embed(q):
        q = jnp.concatenate([zrow, q, zrow], axis=0)
        return jnp.concatenate([zcol, q, zcol], axis=1)

    rowsB = jax.lax.broadcasted_iota(jnp.int32, (BAND, 272), 0) + (r0 - 8)
    colsB = jax.lax.broadcasted_iota(jnp.int32, (BAND, 272), 1) - 8
    inimg = (rowsB >= 0) & (rowsB < H) & (colsB >= 0) & (colsB < W)
    fz = jnp.where(inimg, embed(smooth_q(fpad)), 0.0)
    gz = jnp.where(inimg, embed(smooth_q(gpad)), 0.0)

    c5b = c5b_ref[...]
    c53b = c53b_ref[...]

    def border_int(P):
        hi = jnp.floor(P * (1.0 / 256.0))
        lo = P - hi * 256.0
        mh = jax.lax.dot_general(hi.astype(jnp.bfloat16), c53b, _DN,
                                 preferred_element_type=jnp.float32)
        ml = jax.lax.dot_general(lo.astype(jnp.bfloat16), c53b, _DN,
                                 preferred_element_type=jnp.float32)
        m = mh * 256.0 + ml
        m5 = m[:, 0:256]
        m3 = m[:, 256:512]
        r5 = (m5[0:PR] + m5[1:PR + 1] + m5[2:PR + 2]
              + m5[3:PR + 3] + m5[4:PR + 4])
        r3 = m3[1:PR + 1] + m3[2:PR + 2] + m3[3:PR + 3]
        return r5 - r3

    Xc = fz[5:5 + QR, :]
    Xd = fz[6:6 + QR, :]
    Xu = fz[4:4 + QR, :]
    rowsQ = jax.lax.broadcasted_iota(jnp.int32, (QR, 272), 0) + (r0 - 3)
    colsQ = jax.lax.broadcasted_iota(jnp.int32, (QR, 272), 1) - 8
    dfy = jnp.where(rowsQ == H - 1, Xc, Xd) - jnp.where(rowsQ == 0, Xc, Xu)
    Xr = jnp.concatenate([Xc[:, 1:], Xc[:, 271:272]], axis=1)
    Xl = jnp.concatenate([Xc[:, 0:1], Xc[:, 0:271]], axis=1)
    dfx = jnp.where(colsQ == W - 1, Xc, Xr) - jnp.where(colsQ == 0, Xc, Xl)
    qmask = (rowsQ >= 0) & (rowsQ < H) & (colsQ >= 0) & (colsQ < W)
    dfy = jnp.where(qmask, dfy, 0.0)
    dfx = jnp.where(qmask, dfx, 0.0)
    DX = dfx[:, 6:266]
    DY = dfy[:, 6:266]

    F2 = fz[5:5 + QR, 6:266]

    gzb = gz.astype(jnp.bfloat16)
    F2b = F2.astype(jnp.bfloat16)

    bc = vy = vx = Px = Py = None
    for (jy, jx) in _SPIRAL:
        Gq = gz[5 + jy:5 + QR + jy, 6 + jx:266 + jx]
        E = jnp.abs(gzb[5 + jy:5 + QR + jy, 6 + jx:266 + jx] - F2b)
        c5 = jax.lax.dot_general(E, c5b, _DN,
                                 preferred_element_type=jnp.float32)
        cost = (c5[0:PR] + c5[1:PR + 1] + c5[2:PR + 2]
                + c5[3:PR + 3] + c5[4:PR + 4])
        px = border_int(Gq * DX)
        py = border_int(Gq * DY)
        if bc is None:
            bc = cost
            vy = jnp.zeros_like(cost)
            vx = jnp.zeros_like(cost)
            Px = px
            Py = py
        else:
            m = cost < bc
            bc = jnp.where(m, cost, bc)
            vy = jnp.where(m, float(-jy), vy)
            vx = jnp.where(m, float(-jx), vx)
            Px = jnp.where(m, px, Px)
            Py = jnp.where(m, py, Py)

    gsc = 1.0 / (510.0 * 510.0)
    psc = 1.0 / (255.0 * 510.0)
    A = border_int(DX * DX) * gsc
    Bv = border_int(DX * DY) * gsc
    Dv = border_int(DY * DY) * gsc
    Fx = border_int(F2 * DX)
    Fy = border_int(F2 * DY)
    p = (Px - Fx) * psc
    q = (Py - Fy) * psc
    det = A * Dv - Bv * Bv
    bad = det <= 1e-7
    sd = jnp.where(bad, 1.0, det)
    u = (Dv * p - Bv * q) / sd
    v = (A * q - Bv * p) / sd
    u = jnp.where(bad | (jnp.abs(u) >= 1.0), 0.0, u)
    v = jnp.where(bad | (jnp.abs(v) >= 1.0), 0.0, v)
    fl0 = vy + v
    fl1 = vx + u

    rowsO = jax.lax.broadcasted_iota(jnp.int32, (STRIP, 256), 0) + r0

    def med3(a, b, c):
        return jnp.maximum(jnp.minimum(a, b), jnp.minimum(jnp.maximum(a, b), c))

    def sort3(a, b, c):
        lo = jnp.minimum(a, b)
        hi = jnp.maximum(a, b)
        mx = jnp.maximum(hi, c)
        m2 = jnp.minimum(hi, c)
        return jnp.minimum(lo, m2), jnp.maximum(lo, m2), mx

    def median9(ch):
        cur = ch[1:1 + STRIP]
        up = jnp.where(rowsO == 0, cur, ch[0:STRIP])
        dn = jnp.where(rowsO == H - 1, cur, ch[2:2 + STRIP])

        def lr(t):
            L = jnp.concatenate([t[:, 0:1], t[:, 0:255]], axis=1)
            R = jnp.concatenate([t[:, 1:256], t[:, 255:256]], axis=1)
            return L, t, R

        mn0, md0, mx0 = sort3(*lr(up))
        mn1, md1, mx1 = sort3(*lr(cur))
        mn2, md2, mx2 = sort3(*lr(dn))
        return med3(jnp.maximum(jnp.maximum(mn0, mn1), mn2),
                    med3(md0, md1, md2),
                    jnp.minimum(jnp.minimum(mx0, mx1), mx2))

    out_ref[0, 0, :, :] = median9(fl0)
    out_ref[0, 1, :, :] = median9(fl1)


def _run(fp, gp, interpret=False):
    c5b = jnp.asarray(_C5_NP, jnp.bfloat16)
    c53b = jnp.asarray(_C53_NP, jnp.bfloat16)
    full = lambda shape: pl.BlockSpec(shape, lambda i: tuple(0 for _ in shape))
    return pl.pallas_call(
        _flow_kernel,
        grid=(NSTRIP,),
        in_specs=[full((PH, PH)), full((PH, PH)),
                  full((260, 256)), full((260, 512))],
        out_specs=pl.BlockSpec((1, 2, STRIP, W), lambda i: (0, 0, i, 0)),
        out_shape=jax.ShapeDtypeStruct((1, 2, H, W), jnp.float32),
        interpret=interpret,
    )(fp, gp, c5b, c53b)


def kernel(f, g):
    fp = jnp.pad(f[0, 0], PAD, mode='edge')
    gp = jnp.pad(g[0, 0], PAD, mode='edge')
    return _run(fp, gp)


# speedup vs baseline: 1.3890x; 1.3890x over previous
import numpy as np
import jax
import jax.numpy as jnp
from jax.experimental import pallas as pl

H = 256
W = 256
STRIP = 64
NSTRIP = H // STRIP
PAD = 8
PH = H + 2 * PAD
BAND = STRIP + 2 * PAD
PR = STRIP + 2
QR = PR + 4

def _spiral_coords(sr):
    coords = [(0, 0)]
    y = x = 0
    moves = [(0, 1), (1, 0), (0, -1), (-1, 0)]
    step = 1
    d = 0
    sz = 2 * sr + 1
    while len(coords) < sz * sz:
        for _ in range(2):
            dy, dx = moves[d % 4]
            for _ in range(step):
                y += dy
                x += dx
                if abs(y) <= sr and abs(x) <= sr and len(coords) < sz * sz:
                    coords.append((y, x))
            d += 1
        step += 1
    return coords


_SPIRAL = _spiral_coords(3)

_C5_NP = np.zeros((260, 256), np.float32)
_C3_NP = np.zeros((260, 256), np.float32)
for _i in range(260):
    for _j in range(256):
        if 0 <= _i - _j <= 4:
            _C5_NP[_i, _j] = 1.0
        if 1 <= _i - _j <= 3:
            _C3_NP[_i, _j] = 1.0
_C53_NP = np.concatenate([_C5_NP, _C3_NP], axis=1)

_DN = (((1,), (0,)), ((), ()))


def _flow_kernel(fp_ref, gp_ref, c5b_ref, c53b_ref, out_ref):
    pid = pl.program_id(0)
    r0 = pid * STRIP
    fpad = fp_ref[pl.ds(r0, BAND), :]
    gpad = gp_ref[pl.ds(r0, BAND), :]

    def smooth_q(x):
        v = (x[0:BAND - 2, :] + 2.0 * x[1:BAND - 1, :] + x[2:BAND, :]) * 0.25
        h = (v[:, 0:270] + 2.0 * v[:, 1:271] + v[:, 2:272]) * 0.25
        return jnp.clip(jnp.round(h * 255.0), 0.0, 255.0)

    zrow = jnp.zeros((1, 270), jnp.float32)
    zcol = jnp.zeros((BAND, 1), jnp.float32)

    def embed(q):
        q = jnp.concatenate([zrow, q, zrow], axis=0)
        return jnp.concatenate([zcol, q, zcol], axis=1)

    rowsB = jax.lax.broadcasted_iota(jnp.int32, (BAND, 272), 0) + (r0 - 8)
    colsB = jax.lax.broadcasted_iota(jnp.int32, (BAND, 272), 1) - 8
    inimg = (rowsB >= 0) & (rowsB < H) & (colsB >= 0) & (colsB < W)
    fz = jnp.where(inimg, embed(smooth_q(fpad)), 0.0)
    gz = jnp.where(inimg, embed(smooth_q(gpad)), 0.0)

    c5b = c5b_ref[...]
    c53b = c53b_ref[...]

    def border_int(P):
        hi = jnp.floor(P * (1.0 / 256.0))
        lo = P - hi * 256.0
        mh = jax.lax.dot_general(hi, c53b, _DN,
                                 preferred_element_type=jnp.float32)
        ml = jax.lax.dot_general(lo, c53b, _DN,
                                 preferred_element_type=jnp.float32)
        m = mh * 256.0 + ml
        m5 = m[:, 0:256]
        m3 = m[:, 256:512]
        r5 = (m5[0:PR] + m5[1:PR + 1] + m5[2:PR + 2]
              + m5[3:PR + 3] + m5[4:PR + 4])
        r3 = m3[1:PR + 1] + m3[2:PR + 2] + m3[3:PR + 3]
        return r5 - r3

    Xc = fz[5:5 + QR, :]
    Xd = fz[6:6 + QR, :]
    Xu = fz[4:4 + QR, :]
    rowsQ = jax.lax.broadcasted_iota(jnp.int32, (QR, 272), 0) + (r0 - 3)
    colsQ = jax.lax.broadcasted_iota(jnp.int32, (QR, 272), 1) - 8
    dfy = jnp.where(rowsQ == H - 1, Xc, Xd) - jnp.where(rowsQ == 0, Xc, Xu)
    Xr = jnp.concatenate([Xc[:, 1:], Xc[:, 271:272]], axis=1)
    Xl = jnp.concatenate([Xc[:, 0:1], Xc[:, 0:271]], axis=1)
    dfx = jnp.where(colsQ == W - 1, Xc, Xr) - jnp.where(colsQ == 0, Xc, Xl)
    qmask = (rowsQ >= 0) & (rowsQ < H) & (colsQ >= 0) & (colsQ < W)
    dfy = jnp.where(qmask, dfy, 0.0)
    dfx = jnp.where(qmask, dfx, 0.0)
    DX = dfx[:, 6:266]
    DY = dfy[:, 6:266]

    F2 = fz[5:5 + QR, 6:266]

    gzb = gz.astype(jnp.bfloat16)
    F2b = F2.astype(jnp.bfloat16)

    bc = vy = vx = Px = Py = None
    for (jy, jx) in _SPIRAL:
        Gq = gz[5 + jy:5 + QR + jy, 6 + jx:266 + jx]
        E = jnp.abs(gzb[5 + jy:5 + QR + jy, 6 + jx:266 + jx] - F2b)
        c5 = jax.lax.dot_general(E, c5b, _DN,
                                 preferred_element_type=jnp.float32)
        cost = (c5[0:PR] + c5[1:PR + 1] + c5[2:PR + 2]
                + c5[3:PR + 3] + c5[4:PR + 4])
        px = border_int(Gq * DX)
        py = border_int(Gq * DY)
        if bc is None:
            bc = cost
            vy = jnp.zeros_like(cost)
            vx = jnp.zeros_like(cost)
            Px = px
            Py = py
        else:
            m = cost < bc
            bc = jnp.where(m, cost, bc)
            vy = jnp.where(m, float(-jy), vy)
            vx = jnp.where(m, float(-jx), vx)
            Px = jnp.where(m, px, Px)
            Py = jnp.where(m, py, Py)

    gsc = 1.0 / (510.0 * 510.0)
    psc = 1.0 / (255.0 * 510.0)
    A = border_int(DX * DX) * gsc
    Bv = border_int(DX * DY) * gsc
    Dv = border_int(DY * DY) * gsc
    Fx = border_int(F2 * DX)
    Fy = border_int(F2 * DY)
    p = (Px - Fx) * psc
    q = (Py - Fy) * psc
    det = A * Dv - Bv * Bv
    bad = det <= 1e-7
    sd = jnp.where(bad, 1.0, det)
    u = (Dv * p - Bv * q) / sd
    v = (A * q - Bv * p) / sd
    u = jnp.where(bad | (jnp.abs(u) >= 1.0), 0.0, u)
    v = jnp.where(bad | (jnp.abs(v) >= 1.0), 0.0, v)
    fl0 = vy + v
    fl1 = vx + u

    rowsO = jax.lax.broadcasted_iota(jnp.int32, (STRIP, 256), 0) + r0

    def med3(a, b, c):
        return jnp.maximum(jnp.minimum(a, b), jnp.minimum(jnp.maximum(a, b), c))

    def sort3(a, b, c):
        lo = jnp.minimum(a, b)
        hi = jnp.maximum(a, b)
        mx = jnp.maximum(hi, c)
        m2 = jnp.minimum(hi, c)
        return jnp.minimum(lo, m2), jnp.maximum(lo, m2), mx

    def median9(ch):
        cur = ch[1:1 + STRIP]
        up = jnp.where(rowsO == 0, cur, ch[0:STRIP])
        dn = jnp.where(rowsO == H - 1, cur, ch[2:2 + STRIP])

        def lr(t):
            L = jnp.concatenate([t[:, 0:1], t[:, 0:255]], axis=1)
            R = jnp.concatenate([t[:, 1:256], t[:, 255:256]], axis=1)
            return L, t, R

        mn0, md0, mx0 = sort3(*lr(up))
        mn1, md1, mx1 = sort3(*lr(cur))
        mn2, md2, mx2 = sort3(*lr(dn))
        return med3(jnp.maximum(jnp.maximum(mn0, mn1), mn2),
                    med3(md0, md1, md2),
                    jnp.minimum(jnp.minimum(mx0, mx1), mx2))

    out_ref[0, 0, :, :] = median9(fl0)
    out_ref[0, 1, :, :] = median9(fl1)


def _run(fp, gp, interpret=False):
    c5b = jnp.asarray(_C5_NP, jnp.bfloat16)
    c53b = jnp.asarray(_C53_NP)
    full = lambda shape: pl.BlockSpec(shape, lambda i: tuple(0 for _ in shape))
    return pl.pallas_call(
        _flow_kernel,
        grid=(NSTRIP,),
        in_specs=[full((PH, PH)), full((PH, PH)),
                  full((260, 256)), full((260, 512))],
        out_specs=pl.BlockSpec((1, 2, STRIP, W), lambda i: (0, 0, i, 0)),
        out_shape=jax.ShapeDtypeStruct((1, 2, H, W), jnp.float32),
        interpret=interpret,
    )(fp, gp, c5b, c53b)


def kernel(f, g):
    fp = jnp.pad(f[0, 0], PAD, mode='edge')
    gp = jnp.pad(g[0, 0], PAD, mode='edge')
    return _run(fp, gp)


# speedup vs baseline: 1.4081x; 1.0138x over previous
import numpy as np
import jax
import jax.numpy as jnp
from jax.experimental import pallas as pl

H = 256
W = 256
STRIP = 64
NSTRIP = H // STRIP
PAD = 8
PH = H + 2 * PAD
BAND = STRIP + 2 * PAD
PR = STRIP + 2
QR = PR + 4

def _spiral_coords(sr):
    coords = [(0, 0)]
    y = x = 0
    moves = [(0, 1), (1, 0), (0, -1), (-1, 0)]
    step = 1
    d = 0
    sz = 2 * sr + 1
    while len(coords) < sz * sz:
        for _ in range(2):
            dy, dx = moves[d % 4]
            for _ in range(step):
                y += dy
                x += dx
                if abs(y) <= sr and abs(x) <= sr and len(coords) < sz * sz:
                    coords.append((y, x))
            d += 1
        step += 1
    return coords


_SPIRAL = _spiral_coords(3)

_C5_NP = np.zeros((260, 256), np.float32)
_C3_NP = np.zeros((260, 256), np.float32)
for _i in range(260):
    for _j in range(256):
        if 0 <= _i - _j <= 4:
            _C5_NP[_i, _j] = 1.0
        if 1 <= _i - _j <= 3:
            _C3_NP[_i, _j] = 1.0
_C53_NP = np.concatenate([_C5_NP, _C3_NP], axis=1)

_DN = (((1,), (0,)), ((), ()))


def _flow_kernel(fp_ref, gp_ref, c5b_ref, c53b_ref, out_ref):
    pid = pl.program_id(0)
    r0 = pid * STRIP
    fpad = fp_ref[pl.ds(r0, BAND), :]
    gpad = gp_ref[pl.ds(r0, BAND), :]

    def smooth_q(x):
        v = (x[0:BAND - 2, :] + 2.0 * x[1:BAND - 1, :] + x[2:BAND, :]) * 0.25
        h = (v[:, 0:270] + 2.0 * v[:, 1:271] + v[:, 2:272]) * 0.25
        return jnp.clip(jnp.round(h * 255.0), 0.0, 255.0)

    zrow = jnp.zeros((1, 270), jnp.float32)
    zcol = jnp.zeros((BAND, 1), jnp.float32)

    def embed(q):
        q = jnp.concatenate([zrow, q, zrow], axis=0)
        return jnp.concatenate([zcol, q, zcol], axis=1)

    rowsB = jax.lax.broadcasted_iota(jnp.int32, (BAND, 272), 0) + (r0 - 8)
    colsB = jax.lax.broadcasted_iota(jnp.int32, (BAND, 272), 1) - 8
    inimg = (rowsB >= 0) & (rowsB < H) & (colsB >= 0) & (colsB < W)
    fz = jnp.where(inimg, embed(smooth_q(fpad)), 0.0)
    gz = jnp.where(inimg, embed(smooth_q(gpad)), 0.0)

    c5b = c5b_ref[...]
    c53b = c53b_ref[...]

    def border_int(P):
        hi = jnp.floor(P * (1.0 / 256.0))
        lo = P - hi * 256.0
        mh = jax.lax.dot_general(hi, c53b, _DN,
                                 preferred_element_type=jnp.float32)
        ml = jax.lax.dot_general(lo, c53b, _DN,
                                 preferred_element_type=jnp.float32)
        m = mh * 256.0 + ml
        m5 = m[:, 0:256]
        m3 = m[:, 256:512]
        r5 = (m5[0:PR] + m5[1:PR + 1] + m5[2:PR + 2]
              + m5[3:PR + 3] + m5[4:PR + 4])
        r3 = m3[1:PR + 1] + m3[2:PR + 2] + m3[3:PR + 3]
        return r5 - r3

    Xc = fz[5:5 + QR, :]
    Xd = fz[6:6 + QR, :]
    Xu = fz[4:4 + QR, :]
    rowsQ = jax.lax.broadcasted_iota(jnp.int32, (QR, 272), 0) + (r0 - 3)
    colsQ = jax.lax.broadcasted_iota(jnp.int32, (QR, 272), 1) - 8
    dfy = jnp.where(rowsQ == H - 1, Xc, Xd) - jnp.where(rowsQ == 0, Xc, Xu)
    Xr = jnp.concatenate([Xc[:, 1:], Xc[:, 271:272]], axis=1)
    Xl = jnp.concatenate([Xc[:, 0:1], Xc[:, 0:271]], axis=1)
    dfx = jnp.where(colsQ == W - 1, Xc, Xr) - jnp.where(colsQ == 0, Xc, Xl)
    qmask = (rowsQ >= 0) & (rowsQ < H) & (colsQ >= 0) & (colsQ < W)
    dfy = jnp.where(qmask, dfy, 0.0)
    dfx = jnp.where(qmask, dfx, 0.0)
    DX = dfx[:, 6:266]
    DY = dfy[:, 6:266]

    F2 = fz[5:5 + QR, 6:266]

    gzb = gz.astype(jnp.bfloat16)
    F2b = F2.astype(jnp.bfloat16)

    bc = bi = Px = Py = None
    for (jy, jx) in _SPIRAL:
        Gq = gz[5 + jy:5 + QR + jy, 6 + jx:266 + jx]
        E = jnp.abs(gzb[5 + jy:5 + QR + jy, 6 + jx:266 + jx] - F2b)
        c5 = jax.lax.dot_general(E, c5b, _DN,
                                 preferred_element_type=jnp.float32)
        cost = (c5[0:PR] + c5[1:PR + 1] + c5[2:PR + 2]
                + c5[3:PR + 3] + c5[4:PR + 4])
        px = border_int(Gq * DX)
        py = border_int(Gq * DY)
        code = (jy + 3) * 7 + (jx + 3)
        if bc is None:
            bc = cost
            bi = jnp.full_like(cost, code, dtype=jnp.int32)
            Px = px
            Py = py
        else:
            m = cost < bc
            bc = jnp.where(m, cost, bc)
            bi = jnp.where(m, code, bi)
            Px = jnp.where(m, px, Px)
            Py = jnp.where(m, py, Py)

    gsc = 1.0 / (510.0 * 510.0)
    psc = 1.0 / (255.0 * 510.0)
    A = border_int(DX * DX) * gsc
    Bv = border_int(DX * DY) * gsc
    Dv = border_int(DY * DY) * gsc
    Fx = border_int(F2 * DX)
    Fy = border_int(F2 * DY)
    p = (Px - Fx) * psc
    q = (Py - Fy) * psc
    det = A * Dv - Bv * Bv
    bad = det <= 1e-7
    sd = jnp.where(bad, 1.0, det)
    u = (Dv * p - Bv * q) / sd
    v = (A * q - Bv * p) / sd
    u = jnp.where(bad | (jnp.abs(u) >= 1.0), 0.0, u)
    v = jnp.where(bad | (jnp.abs(v) >= 1.0), 0.0, v)
    jyp = jax.lax.shift_right_logical(bi * 9363, 16)
    jxp = bi - jyp * 7
    fl0 = (3 - jyp).astype(jnp.float32) + v
    fl1 = (3 - jxp).astype(jnp.float32) + u

    rowsO = jax.lax.broadcasted_iota(jnp.int32, (STRIP, 256), 0) + r0

    def med3(a, b, c):
        return jnp.maximum(jnp.minimum(a, b), jnp.minimum(jnp.maximum(a, b), c))

    def sort3(a, b, c):
        lo = jnp.minimum(a, b)
        hi = jnp.maximum(a, b)
        mx = jnp.maximum(hi, c)
        m2 = jnp.minimum(hi, c)
        return jnp.minimum(lo, m2), jnp.maximum(lo, m2), mx

    def median9(ch):
        cur = ch[1:1 + STRIP]
        up = jnp.where(rowsO == 0, cur, ch[0:STRIP])
        dn = jnp.where(rowsO == H - 1, cur, ch[2:2 + STRIP])

        def lr(t):
            L = jnp.concatenate([t[:, 0:1], t[:, 0:255]], axis=1)
            R = jnp.concatenate([t[:, 1:256], t[:, 255:256]], axis=1)
            return L, t, R

        mn0, md0, mx0 = sort3(*lr(up))
        mn1, md1, mx1 = sort3(*lr(cur))
        mn2, md2, mx2 = sort3(*lr(dn))
        return med3(jnp.maximum(jnp.maximum(mn0, mn1), mn2),
                    med3(md0, md1, md2),
                    jnp.minimum(jnp.minimum(mx0, mx1), mx2))

    out_ref[0, 0, :, :] = median9(fl0)
    out_ref[0, 1, :, :] = median9(fl1)


def _run(fp, gp, interpret=False):
    c5b = jnp.asarray(_C5_NP, jnp.bfloat16)
    c53b = jnp.asarray(_C53_NP)
    full = lambda shape: pl.BlockSpec(shape, lambda i: tuple(0 for _ in shape))
    return pl.pallas_call(
        _flow_kernel,
        grid=(NSTRIP,),
        in_specs=[full((PH, PH)), full((PH, PH)),
                  full((260, 256)), full((260, 512))],
        out_specs=pl.BlockSpec((1, 2, STRIP, W), lambda i: (0, 0, i, 0)),
        out_shape=jax.ShapeDtypeStruct((1, 2, H, W), jnp.float32),
        interpret=interpret,
    )(fp, gp, c5b, c53b)


def kernel(f, g):
    fp = jnp.pad(f[0, 0], PAD, mode='edge')
    gp = jnp.pad(g[0, 0], PAD, mode='edge')
    return _run(fp, gp)
